# baseline (speedup 1.0000x reference)
import functools

import jax
import jax.numpy as jnp
from jax import lax
from jax.experimental import pallas as pl
from jax.experimental.pallas import tpu as pltpu
from jax.experimental.pallas import tpu_sc as plsc

NC = 2
NS = 16
NW = NC * NS

LANES = 16

VCHUNK = 65536

IDX_PER_STREAM = 64
WAVES_IN_FLIGHT = 8


def _tc_project_pack(embT, Wt, b2, hist):
    E, V = embT.shape
    inv = 1.0 / float(hist)
    grid = (V + VCHUNK - 1) // VCHUNK

    def body(embT_ref, wt_ref, b_ref, o_ref):
        p = jnp.dot(wt_ref[...], embT_ref[...],
                    preferred_element_type=jnp.float32)
        p = (p + b_ref[...]) * inv
        lo = lax.bitcast_convert_type(
            p[0].astype(jnp.bfloat16), jnp.uint16).astype(jnp.uint32)
        hi = lax.bitcast_convert_type(
            p[1].astype(jnp.bfloat16), jnp.uint16).astype(jnp.uint32)
        o_ref[...] = (hi << 16) | lo

    return pl.pallas_call(
        body,
        grid=(grid,),
        in_specs=[
            pl.BlockSpec((E, VCHUNK), lambda i: (0, i)),
            pl.BlockSpec((2, E), lambda i: (0, 0)),
            pl.BlockSpec((2, 1), lambda i: (0, 0)),
        ],
        out_specs=pl.BlockSpec((VCHUNK,), lambda i: (i,)),
        out_shape=jax.ShapeDtypeStruct((V,), jnp.uint32),
    )(embT, Wt, b2)


def _sc_gather_pool(xT, pp, batch, hist):
    batch_per_worker = batch // NW
    rows_per_worker = batch_per_worker * hist
    streams_per_wave = batch_per_worker // IDX_PER_STREAM
    pool_groups = batch_per_worker // LANES
    assert batch_per_worker % IDX_PER_STREAM == 0
    assert batch_per_worker % LANES == 0

    mesh = plsc.VectorSubcoreMesh(
        core_axis_name="c", subcore_axis_name="s",
        num_cores=NC, num_subcores=NS)

    @functools.partial(
        pl.kernel,
        out_type=[
            jax.ShapeDtypeStruct((batch,), jnp.float32),
            jax.ShapeDtypeStruct((batch,), jnp.float32),
        ],
        mesh=mesh,
        scratch_types=[
            pltpu.VMEM((hist, batch_per_worker), jnp.int32),
            pltpu.VMEM((rows_per_worker,), jnp.uint32),
            pltpu.VMEM((batch_per_worker,), jnp.float32),
            pltpu.VMEM((batch_per_worker,), jnp.float32),
            pltpu.SemaphoreType.DMA,
        ],
        compiler_params=pltpu.CompilerParams(
            needs_layout_passes=False, use_tc_tiling_on_sc=False),
    )
    def k(xT_hbm, pp_hbm, out0_hbm, out1_hbm, xv, bv, ob0, ob1, sem):
        wid = lax.axis_index("s") * NC + lax.axis_index("c")
        wslice = pl.ds(wid * batch_per_worker, batch_per_worker)

        pltpu.sync_copy(xT_hbm.at[pl.ds(0, hist), wslice], xv)

        def fire(l):
            for t in range(streams_per_wave):
                pltpu.async_copy(
                    pp_hbm.at[xv.at[l, pl.ds(t * IDX_PER_STREAM,
                                             IDX_PER_STREAM)]],
                    bv.at[pl.ds(l * batch_per_worker + t * IDX_PER_STREAM,
                                IDX_PER_STREAM)],
                    sem)

        def drain_wave():
            pltpu.make_async_copy(
                pp_hbm.at[pl.ds(0, batch_per_worker)],
                bv.at[pl.ds(0, batch_per_worker)], sem).wait()

        zero = jnp.zeros((LANES,), jnp.float32)

        def zero_body(g, carry):
            ob0[pl.ds(g * LANES, LANES)] = zero
            ob1[pl.ds(g * LANES, LANES)] = zero
            return carry

        lax.fori_loop(0, pool_groups, zero_body, 0)

        for l in range(WAVES_IN_FLIGHT):
            fire(l)

        def gather_body(l, carry):
            @pl.when(l + WAVES_IN_FLIGHT < hist)
            def _():
                fire(l + WAVES_IN_FLIGHT)

            drain_wave()

            def acc_body(g, c):
                sl = pl.ds(g * LANES, LANES)
                w = bv[pl.ds(l * batch_per_worker + g * LANES, LANES)]
                e = plsc.bitcast(w << 16, jnp.float32)
                o = plsc.bitcast(w & jnp.uint32(0xFFFF0000), jnp.float32)
                ob0[sl] = ob0[sl] + e
                ob1[sl] = ob1[sl] + o
                return c

            lax.fori_loop(0, pool_groups, acc_body, 0)
            return carry

        lax.fori_loop(0, hist, gather_body, 0)

        pltpu.sync_copy(ob0, out0_hbm.at[wslice])
        pltpu.sync_copy(ob1, out1_hbm.at[wslice])

    return k(xT, pp)


def kernel(x, emb, W, b):
    batch, hist = x.shape
    pp = _tc_project_pack(emb.T, W.T, b.reshape(-1, 1), hist)
    xT = x.astype(jnp.int32).T
    out0, out1 = _sc_gather_pool(xT, pp, batch, hist)
    return jnp.stack([out0, out1], axis=1)


# speedup vs baseline: 1.0018x; 1.0018x over previous
import functools

import jax
import jax.numpy as jnp
from jax import lax
from jax.experimental import pallas as pl
from jax.experimental.pallas import tpu as pltpu
from jax.experimental.pallas import tpu_sc as plsc

NC = 2
NS = 16
NW = NC * NS

LANES = 16

VCHUNK = 65536

IDX_PER_STREAM = 128
WAVES_IN_FLIGHT = 8


def _tc_project_pack(embT, Wt, b2, hist):
    E, V = embT.shape
    inv = 1.0 / float(hist)
    grid = (V + VCHUNK - 1) // VCHUNK

    def body(embT_ref, wt_ref, b_ref, o_ref):
        p = jnp.dot(wt_ref[...], embT_ref[...],
                    preferred_element_type=jnp.float32)
        p = (p + b_ref[...]) * inv
        lo = lax.bitcast_convert_type(
            p[0].astype(jnp.bfloat16), jnp.uint16).astype(jnp.uint32)
        hi = lax.bitcast_convert_type(
            p[1].astype(jnp.bfloat16), jnp.uint16).astype(jnp.uint32)
        o_ref[...] = (hi << 16) | lo

    return pl.pallas_call(
        body,
        grid=(grid,),
        in_specs=[
            pl.BlockSpec((E, VCHUNK), lambda i: (0, i)),
            pl.BlockSpec((2, E), lambda i: (0, 0)),
            pl.BlockSpec((2, 1), lambda i: (0, 0)),
        ],
        out_specs=pl.BlockSpec((VCHUNK,), lambda i: (i,)),
        out_shape=jax.ShapeDtypeStruct((V,), jnp.uint32),
    )(embT, Wt, b2)


def _sc_gather_pool(xT, pp, batch, hist):
    batch_per_worker = batch // NW
    rows_per_worker = batch_per_worker * hist
    streams_per_wave = batch_per_worker // IDX_PER_STREAM
    pool_groups = batch_per_worker // LANES
    assert batch_per_worker % IDX_PER_STREAM == 0
    assert batch_per_worker % LANES == 0

    mesh = plsc.VectorSubcoreMesh(
        core_axis_name="c", subcore_axis_name="s",
        num_cores=NC, num_subcores=NS)

    @functools.partial(
        pl.kernel,
        out_type=[
            jax.ShapeDtypeStruct((batch,), jnp.float32),
            jax.ShapeDtypeStruct((batch,), jnp.float32),
        ],
        mesh=mesh,
        scratch_types=[
            pltpu.VMEM((hist, batch_per_worker), jnp.int32),
            pltpu.VMEM((rows_per_worker,), jnp.uint32),
            pltpu.VMEM((batch_per_worker,), jnp.float32),
            pltpu.VMEM((batch_per_worker,), jnp.float32),
            pltpu.SemaphoreType.DMA,
        ],
        compiler_params=pltpu.CompilerParams(
            needs_layout_passes=False, use_tc_tiling_on_sc=False),
    )
    def k(xT_hbm, pp_hbm, out0_hbm, out1_hbm, xv, bv, ob0, ob1, sem):
        wid = lax.axis_index("s") * NC + lax.axis_index("c")
        wslice = pl.ds(wid * batch_per_worker, batch_per_worker)

        pltpu.sync_copy(xT_hbm.at[pl.ds(0, hist), wslice], xv)

        def fire(l):
            for t in range(streams_per_wave):
                pltpu.async_copy(
                    pp_hbm.at[xv.at[l, pl.ds(t * IDX_PER_STREAM,
                                             IDX_PER_STREAM)]],
                    bv.at[pl.ds(l * batch_per_worker + t * IDX_PER_STREAM,
                                IDX_PER_STREAM)],
                    sem)

        def drain_wave():
            pltpu.make_async_copy(
                pp_hbm.at[pl.ds(0, batch_per_worker)],
                bv.at[pl.ds(0, batch_per_worker)], sem).wait()

        zero = jnp.zeros((LANES,), jnp.float32)

        def zero_body(g, carry):
            ob0[pl.ds(g * LANES, LANES)] = zero
            ob1[pl.ds(g * LANES, LANES)] = zero
            return carry

        lax.fori_loop(0, pool_groups, zero_body, 0)

        for l in range(WAVES_IN_FLIGHT):
            fire(l)

        def gather_body(l, carry):
            @pl.when(l + WAVES_IN_FLIGHT < hist)
            def _():
                fire(l + WAVES_IN_FLIGHT)

            drain_wave()

            def acc_body(g, c):
                sl = pl.ds(g * LANES, LANES)
                w = bv[pl.ds(l * batch_per_worker + g * LANES, LANES)]
                e = plsc.bitcast(w << 16, jnp.float32)
                o = plsc.bitcast(w & jnp.uint32(0xFFFF0000), jnp.float32)
                ob0[sl] = ob0[sl] + e
                ob1[sl] = ob1[sl] + o
                return c

            lax.fori_loop(0, pool_groups, acc_body, 0)
            return carry

        lax.fori_loop(0, hist, gather_body, 0)

        pltpu.sync_copy(ob0, out0_hbm.at[wslice])
        pltpu.sync_copy(ob1, out1_hbm.at[wslice])

    return k(xT, pp)


def kernel(x, emb, W, b):
    batch, hist = x.shape
    pp = _tc_project_pack(emb.T, W.T, b.reshape(-1, 1), hist)
    xT = x.astype(jnp.int32).T
    out0, out1 = _sc_gather_pool(xT, pp, batch, hist)
    return jnp.stack([out0, out1], axis=1)


# speedup vs baseline: 1.0240x; 1.0222x over previous
import functools

import jax
import jax.numpy as jnp
from jax import lax
from jax.experimental import pallas as pl
from jax.experimental.pallas import tpu as pltpu
from jax.experimental.pallas import tpu_sc as plsc

NC = 2
NS = 16
NW = NC * NS

LANES = 16

VCHUNK = 65536

IDX_PER_STREAM = 128
WAVES_IN_FLIGHT = 8


def _tc_project_pack(embT, Wt, b2, xT, hist):
    E, V = embT.shape
    L, B = xT.shape
    inv = 1.0 / float(hist)
    grid = (V + VCHUNK - 1) // VCHUNK

    def body(embT_ref, wt_ref, b_ref, x_ref, o_ref, xt_ref):
        p = jnp.dot(wt_ref[...], embT_ref[...],
                    preferred_element_type=jnp.float32)
        p = (p + b_ref[...]) * inv
        lo = lax.bitcast_convert_type(
            p[0].astype(jnp.bfloat16), jnp.uint16).astype(jnp.uint32)
        hi = lax.bitcast_convert_type(
            p[1].astype(jnp.bfloat16), jnp.uint16).astype(jnp.uint32)
        o_ref[...] = (hi << 16) | lo

        @pl.when(pl.program_id(0) == 0)
        def _():
            xt_ref[...] = x_ref[...].reshape(L * B)

    return pl.pallas_call(
        body,
        grid=(grid,),
        in_specs=[
            pl.BlockSpec((E, VCHUNK), lambda i: (0, i)),
            pl.BlockSpec((2, E), lambda i: (0, 0)),
            pl.BlockSpec((2, 1), lambda i: (0, 0)),
            pl.BlockSpec((L, B), lambda i: (0, 0)),
        ],
        out_specs=[
            pl.BlockSpec((VCHUNK,), lambda i: (i,)),
            pl.BlockSpec((L * B,), lambda i: (0,)),
        ],
        out_shape=[
            jax.ShapeDtypeStruct((V,), jnp.uint32),
            jax.ShapeDtypeStruct((L * B,), jnp.int32),
        ],
    )(embT, Wt, b2, xT)


def _sc_gather_pool(xt, pp, batch, hist):
    batch_per_worker = batch // NW
    rows_per_worker = batch_per_worker * hist
    streams_per_wave = batch_per_worker // IDX_PER_STREAM
    pool_groups = batch_per_worker // LANES
    assert batch_per_worker % IDX_PER_STREAM == 0
    assert batch_per_worker % LANES == 0

    mesh = plsc.VectorSubcoreMesh(
        core_axis_name="c", subcore_axis_name="s",
        num_cores=NC, num_subcores=NS)

    @functools.partial(
        pl.kernel,
        out_type=[
            jax.ShapeDtypeStruct((batch,), jnp.float32),
            jax.ShapeDtypeStruct((batch,), jnp.float32),
        ],
        mesh=mesh,
        scratch_types=[
            pltpu.VMEM((hist, batch_per_worker), jnp.int32),
            pltpu.VMEM((rows_per_worker,), jnp.uint32),
            pltpu.VMEM((batch_per_worker,), jnp.float32),
            pltpu.VMEM((batch_per_worker,), jnp.float32),
            pltpu.SemaphoreType.DMA,
        ],
        compiler_params=pltpu.CompilerParams(
            needs_layout_passes=False, use_tc_tiling_on_sc=False),
    )
    def k(xt_hbm, pp_hbm, out0_hbm, out1_hbm, xv, bv, ob0, ob1, sem):
        wid = lax.axis_index("s") * NC + lax.axis_index("c")
        wslice = pl.ds(wid * batch_per_worker, batch_per_worker)

        idx_copies = [
            pltpu.async_copy(
                xt_hbm.at[pl.ds(l * batch + wid * batch_per_worker,
                                batch_per_worker)],
                xv.at[l], sem)
            for l in range(hist)
        ]
        for h in idx_copies:
            h.wait()

        def fire(l):
            for t in range(streams_per_wave):
                pltpu.async_copy(
                    pp_hbm.at[xv.at[l, pl.ds(t * IDX_PER_STREAM,
                                             IDX_PER_STREAM)]],
                    bv.at[pl.ds(l * batch_per_worker + t * IDX_PER_STREAM,
                                IDX_PER_STREAM)],
                    sem)

        def drain_wave():
            pltpu.make_async_copy(
                pp_hbm.at[pl.ds(0, batch_per_worker)],
                bv.at[pl.ds(0, batch_per_worker)], sem).wait()

        zero = jnp.zeros((LANES,), jnp.float32)

        def zero_body(g, carry):
            ob0[pl.ds(g * LANES, LANES)] = zero
            ob1[pl.ds(g * LANES, LANES)] = zero
            return carry

        lax.fori_loop(0, pool_groups, zero_body, 0)

        for l in range(WAVES_IN_FLIGHT):
            fire(l)

        def gather_body(l, carry):
            @pl.when(l + WAVES_IN_FLIGHT < hist)
            def _():
                fire(l + WAVES_IN_FLIGHT)

            drain_wave()

            def acc_body(g, c):
                sl = pl.ds(g * LANES, LANES)
                w = bv[pl.ds(l * batch_per_worker + g * LANES, LANES)]
                e = plsc.bitcast(w << 16, jnp.float32)
                o = plsc.bitcast(w & jnp.uint32(0xFFFF0000), jnp.float32)
                ob0[sl] = ob0[sl] + e
                ob1[sl] = ob1[sl] + o
                return c

            lax.fori_loop(0, pool_groups, acc_body, 0)
            return carry

        lax.fori_loop(0, hist, gather_body, 0)

        pltpu.sync_copy(ob0, out0_hbm.at[wslice])
        pltpu.sync_copy(ob1, out1_hbm.at[wslice])

    return k(xt, pp)


def kernel(x, emb, W, b):
    batch, hist = x.shape
    pp, xt = _tc_project_pack(emb.T, W.T, b.reshape(-1, 1),
                              x.astype(jnp.int32).T, hist)
    out0, out1 = _sc_gather_pool(xt, pp, batch, hist)
    return jnp.stack([out0, out1], axis=1)
